# baseline (speedup 1.0000x reference)
import jax
import jax.numpy as jnp
from jax import lax
from jax.experimental import pallas as pl
from jax.experimental.pallas import tpu as pltpu
from jax.experimental.pallas import tpu_sc as plsc

EMBD = 1024
B = 4
T = 2048

NC = 2
NS = 16
NW = NC * NS

ROWS_PER_W = B * T // NW
TSPAN = T // NW
CHUNK = 8
SUBS = TSPAN // CHUNK
NCHUNK = B * SUBS
NSLOT = 6
VEC = 16
NVEC = EMBD // VEC


def _sc_kernel(x_hbm, word_hbm, pos_hbm, out_hbm, idx_v,
               buf0, buf1, buf2, buf3, buf4, buf5, posb0, posb1,
               gsem, osem, psem, isem):
    wid = lax.axis_index("s") * NC + lax.axis_index("c")
    t0 = wid * TSPAN

    bufs = (buf0, buf1, buf2, buf3, buf4, buf5)
    posb = (posb0, posb1)

    idx_copies = [
        pltpu.async_copy(x_hbm.at[b, pl.ds(t0, TSPAN)],
                         idx_v.at[pl.ds(b * TSPAN, TSPAN)], isem)
        for b in range(B)
    ]

    def chunk_coords(c):
        sub, b = divmod(c, B)
        return b, t0 + sub * CHUNK, b * TSPAN + sub * CHUNK, sub

    def issue_gather(c):
        s = c % NSLOT
        _, _, idx_off, _ = chunk_coords(c)
        return pltpu.async_copy(
            word_hbm.at[idx_v.at[pl.ds(idx_off, CHUNK)]], bufs[s], gsem)

    def issue_pos(sub):
        return pltpu.async_copy(
            pos_hbm.at[pl.ds(t0 + sub * CHUNK, CHUNK)], posb[sub % 2],
            psem)

    waited_idx = set()

    def need_idx(b):
        if b not in waited_idx:
            idx_copies[b].wait()
            waited_idx.add(b)

    pending_g = {}
    pending_p = {}
    need_idx(0)
    pending_g[0] = issue_gather(0)
    pending_p[0] = issue_pos(0)
    for c in range(1, NSLOT - 1):
        need_idx(c % B)
        pending_g[c] = issue_gather(c)
    for b in range(B):
        need_idx(b)
    pending_p[1] = issue_pos(1)
    pending_o = {}

    for c in range(NCHUNK):
        s = c % NSLOT
        b, t_off, _, sub = chunk_coords(c)
        if c % B == 0:
            pending_p.pop(sub).wait()
        pos = posb[sub % 2]
        pending_g.pop(c).wait()
        buf = bufs[s]

        def add_rows(r0, nrows):
            @plsc.parallel_loop(0, nrows * NVEC, 1, unroll=8)
            def vec(n):
                r = r0 + lax.shift_right_logical(n, 6)
                o = pl.multiple_of(
                    lax.shift_left(lax.bitwise_and(n, NVEC - 1), 4), VEC)
                sl = pl.ds(o, VEC)
                plsc.addupdate(buf.at[r, sl], pos[r, sl])

        if c == NCHUNK - 1:
            H = CHUNK // 2
            add_rows(0, H)
            tail_o = [pltpu.async_copy(
                bufs[s].at[pl.ds(0, H)],
                out_hbm.at[b, pl.ds(t_off, H)], osem)]
            add_rows(H, H)
            tail_o.append(pltpu.async_copy(
                bufs[s].at[pl.ds(H, H)],
                out_hbm.at[b, pl.ds(t_off + H, H)], osem))
            pending_o[c] = tail_o
        else:
            add_rows(0, CHUNK)

        if c % B == B - 1 and sub + 2 < SUBS:
            pending_p[sub + 2] = issue_pos(sub + 2)
        if c - 1 in pending_o:
            for o in pending_o.pop(c - 1):
                o.wait()
        if c + NSLOT - 1 < NCHUNK:
            pending_g[c + NSLOT - 1] = issue_gather(c + NSLOT - 1)
        if c != NCHUNK - 1:
            pending_o[c] = [pltpu.async_copy(
                bufs[s], out_hbm.at[b, pl.ds(t_off, CHUNK)], osem)]
    for c in sorted(pending_o):
        for o in pending_o.pop(c):
            o.wait()


@jax.jit
def _run(x, word_table, pos_table):
    mesh = plsc.VectorSubcoreMesh(
        core_axis_name="c", subcore_axis_name="s", num_cores=NC,
        num_subcores=NS,
    )
    return pl.kernel(
        _sc_kernel,
        out_type=jax.ShapeDtypeStruct((B, T, EMBD), jnp.float32),
        mesh=mesh,
        scratch_types=(
            [pltpu.VMEM((ROWS_PER_W,), jnp.int32)]
            + [pltpu.VMEM((CHUNK, EMBD), jnp.float32)] * (NSLOT + 2)
            + [pltpu.SemaphoreType.DMA] * 4
        ),
    )(x, word_table, pos_table)


def kernel(x, word_table, pos_table):
    return _run(x.astype(jnp.int32), word_table, pos_table)


# speedup vs baseline: 1.0299x; 1.0299x over previous
import jax
import jax.numpy as jnp
from jax import lax
from jax.experimental import pallas as pl
from jax.experimental.pallas import tpu as pltpu
from jax.experimental.pallas import tpu_sc as plsc

EMBD = 1024
B = 4
T = 2048

NC = 2
NS = 16
NW = NC * NS

ROWS_PER_W = B * T // NW
TSPAN = T // NW
CHUNK = 16
SUBS = TSPAN // CHUNK
NCHUNK = B * SUBS
NSLOT = 4
VEC = 16
NVEC = EMBD // VEC


def _sc_kernel(x_hbm, word_hbm, pos_hbm, out_hbm, idx_v,
               buf0, buf1, buf2, buf3, posb0, posb1,
               gsem, osem, psem, isem):
    wid = lax.axis_index("s") * NC + lax.axis_index("c")
    t0 = wid * TSPAN

    bufs = (buf0, buf1, buf2, buf3)
    posb = (posb0, posb1)

    idx_copies = [
        pltpu.async_copy(x_hbm.at[b, pl.ds(t0, TSPAN)],
                         idx_v.at[pl.ds(b * TSPAN, TSPAN)], isem)
        for b in range(B)
    ]

    def chunk_coords(c):
        sub, b = divmod(c, B)
        return b, t0 + sub * CHUNK, b * TSPAN + sub * CHUNK, sub

    def issue_gather(c):
        s = c % NSLOT
        _, _, idx_off, _ = chunk_coords(c)
        return pltpu.async_copy(
            word_hbm.at[idx_v.at[pl.ds(idx_off, CHUNK)]], bufs[s], gsem)

    def issue_pos(sub):
        return pltpu.async_copy(
            pos_hbm.at[pl.ds(t0 + sub * CHUNK, CHUNK)], posb[sub % 2],
            psem)

    pending_g = {}
    pending_p = {}
    idx_copies[0].wait()
    pending_g[0] = issue_gather(0)
    pending_p[0] = issue_pos(0)
    for c in range(1, NSLOT - 1):
        idx_copies[c].wait()
        pending_g[c] = issue_gather(c)
    idx_copies[B - 1].wait()
    pending_p[1] = issue_pos(1)
    pending_o = {}

    for c in range(NCHUNK):
        s = c % NSLOT
        b, t_off, _, sub = chunk_coords(c)
        if c % B == 0:
            pending_p.pop(sub).wait()
        pos = posb[sub % 2]
        pending_g.pop(c).wait()
        buf = bufs[s]

        def add_rows(r0, nrows):
            @plsc.parallel_loop(0, nrows * NVEC, 1, unroll=8)
            def vec(n):
                r = r0 + lax.shift_right_logical(n, 6)
                o = pl.multiple_of(
                    lax.shift_left(lax.bitwise_and(n, NVEC - 1), 4), VEC)
                sl = pl.ds(o, VEC)
                plsc.addupdate(buf.at[r, sl], pos[r, sl])

        if c == NCHUNK - 1:
            H = CHUNK // 2
            add_rows(0, H)
            tail_o = [pltpu.async_copy(
                bufs[s].at[pl.ds(0, H)],
                out_hbm.at[b, pl.ds(t_off, H)], osem)]
            add_rows(H, H)
            tail_o.append(pltpu.async_copy(
                bufs[s].at[pl.ds(H, H)],
                out_hbm.at[b, pl.ds(t_off + H, H)], osem))
            pending_o[c] = tail_o
        else:
            add_rows(0, CHUNK)

        if c % B == B - 1 and sub + 2 < SUBS:
            pending_p[sub + 2] = issue_pos(sub + 2)
        if c - 1 in pending_o:
            for o in pending_o.pop(c - 1):
                o.wait()
        if c + NSLOT - 1 < NCHUNK:
            pending_g[c + NSLOT - 1] = issue_gather(c + NSLOT - 1)
        if c != NCHUNK - 1:
            pending_o[c] = [pltpu.async_copy(
                bufs[s], out_hbm.at[b, pl.ds(t_off, CHUNK)], osem)]
    for c in sorted(pending_o):
        for o in pending_o.pop(c):
            o.wait()


@jax.jit
def _run(x, word_table, pos_table):
    mesh = plsc.VectorSubcoreMesh(
        core_axis_name="c", subcore_axis_name="s", num_cores=NC,
        num_subcores=NS,
    )
    return pl.kernel(
        _sc_kernel,
        out_type=jax.ShapeDtypeStruct((B, T, EMBD), jnp.float32),
        mesh=mesh,
        scratch_types=(
            [pltpu.VMEM((ROWS_PER_W,), jnp.int32)]
            + [pltpu.VMEM((CHUNK, EMBD), jnp.float32)] * (NSLOT + 2)
            + [pltpu.SemaphoreType.DMA] * 4
        ),
    )(x, word_table, pos_table)


def kernel(x, word_table, pos_table):
    return _run(x.astype(jnp.int32), word_table, pos_table)
